# baseline (speedup 1.0000x reference)
import jax
import jax.numpy as jnp
from jax import lax
from jax.experimental import pallas as pl
from jax.experimental.pallas import tpu as pltpu
from jax.experimental.pallas import tpu_sc as plsc

N_TOP = 100000
N_CLS = 128
B = 16384

NUM_CORES = 2
NUM_SUBCORES = 16
NUM_WORKERS = NUM_CORES * NUM_SUBCORES
LANES = 16
CHUNK = 64
ROWS_PER_W = B // NUM_WORKERS
N_CHUNKS = ROWS_PER_W // CHUNK
SCHUNK = 128
N_SCHUNKS = ROWS_PER_W // SCHUNK
N_LB = 4


def _calib_body(logits_hbm, topics_hbm, loga_hbm, b_hbm, out_hbm,
                idx_v, scale_v,
                rows0, rows1, rows2, rows3, rows4, rows5, rows6, rows7,
                logits0, logits1, logits2, logits3,
                sem_r0, sem_r1, sem_r2, sem_r3,
                sem_r4, sem_r5, sem_r6, sem_r7,
                sem_s0, sem_s1, sem_s2, sem_s3,
                sem_l0, sem_l1, sem_l2, sem_l3,
                sem_o0, sem_o1, sem_o2, sem_o3,
                sem_o4, sem_o5, sem_o6, sem_o7):
    wid = lax.axis_index("s") * NUM_CORES + lax.axis_index("c")
    base = wid * ROWS_PER_W

    rows = (rows0, rows1, rows2, rows3, rows4, rows5, rows6, rows7)
    logits_b = (logits0, logits1, logits2, logits3)
    sem_r = (sem_r0, sem_r1, sem_r2, sem_r3, sem_r4, sem_r5, sem_r6, sem_r7)
    sem_s = (sem_s0, sem_s1, sem_s2, sem_s3)
    sem_l = (sem_l0, sem_l1, sem_l2, sem_l3)
    sem_o = (sem_o0, sem_o1, sem_o2, sem_o3, sem_o4, sem_o5, sem_o6, sem_o7)

    def off(c):
        return pl.ds(pl.multiple_of(base + c * CHUNK, CHUNK), CHUNK)

    def issue_rows(c):
        return pltpu.async_copy(b_hbm.at[idx_v.at[pl.ds(c * CHUNK, CHUNK)]],
                                rows[c], sem_r[c])

    def issue_logits(c):
        return pltpu.async_copy(logits_hbm.at[off(c)], logits_b[c % N_LB],
                                sem_l[c % N_LB])

    pltpu.sync_copy(topics_hbm.at[pl.ds(pl.multiple_of(base, CHUNK), CHUNK)],
                    idx_v.at[pl.ds(0, CHUNK)])
    rows_cp = [None] * N_CHUNKS
    logits_cp = [None] * N_CHUNKS
    rows_cp[0] = issue_rows(0)
    logits_cp[0] = issue_logits(0)
    pltpu.sync_copy(topics_hbm.at[pl.ds(pl.multiple_of(base + CHUNK, CHUNK),
                                        ROWS_PER_W - CHUNK)],
                    idx_v.at[pl.ds(CHUNK, ROWS_PER_W - CHUNK)])
    scale_cp = [pltpu.async_copy(loga_hbm.at[idx_v.at[pl.ds(c * SCHUNK, SCHUNK)]],
                                 scale_v.at[pl.ds(c * SCHUNK, SCHUNK)], sem_s[c])
                for c in range(N_SCHUNKS)]
    for c in range(1, N_CHUNKS):
        rows_cp[c] = issue_rows(c)
    for c in range(1, N_LB):
        logits_cp[c] = issue_logits(c)

    out_cp = [None] * N_CHUNKS
    for c in range(N_CHUNKS):
        if c % 2 == 0:
            scale_cp[c // 2].wait()
            for j in range(SCHUNK // LANES):
                sl = pl.ds(c * CHUNK + j * LANES, LANES)
                scale_v[sl] = jnp.exp(scale_v[sl])
        rows_cp[c].wait()
        logits_cp[c].wait()
        lb = logits_b[c % N_LB]

        def group_body(g, _, c=c, lb=lb):
            sv = scale_v[pl.ds(c * CHUNK + g * LANES, LANES)]
            for rr in range(LANES):
                i = g * LANES + rr
                sc = sv[rr]
                for k in range(N_CLS // LANES):
                    sl = pl.ds(k * LANES, LANES)
                    plsc.addupdate(rows[c].at[i, sl], lb[i, sl] * sc)
            return 0

        lax.fori_loop(0, CHUNK // LANES, group_body, 0)
        out_cp[c] = pltpu.async_copy(rows[c], out_hbm.at[off(c)], sem_o[c])
        if c + N_LB < N_CHUNKS:
            logits_cp[c + N_LB] = issue_logits(c + N_LB)

    for c in range(N_CHUNKS):
        out_cp[c].wait()


@jax.jit
def kernel(logits, topics, loga, b):
    topics = topics.astype(jnp.int32)
    run = pl.kernel(
        _calib_body,
        out_type=jax.ShapeDtypeStruct((B, N_CLS), jnp.float32),
        mesh=plsc.VectorSubcoreMesh(core_axis_name="c", subcore_axis_name="s"),
        scratch_types=[
            pltpu.VMEM((ROWS_PER_W,), jnp.int32),
            pltpu.VMEM((ROWS_PER_W,), jnp.float32),
        ] + [pltpu.VMEM((CHUNK, N_CLS), jnp.float32)] * (N_CHUNKS + N_LB)
          + [pltpu.SemaphoreType.DMA] * 24,
    )
    return run(logits, topics, loga, b)


# speedup vs baseline: 1.2058x; 1.2058x over previous
import jax
import jax.numpy as jnp
from jax import lax
from jax.experimental import pallas as pl
from jax.experimental.pallas import tpu as pltpu
from jax.experimental.pallas import tpu_sc as plsc

N_TOP = 100000
N_CLS = 128
B = 16384

NUM_CORES = 2
NUM_SUBCORES = 16
NUM_WORKERS = NUM_CORES * NUM_SUBCORES
LANES = 16
CHUNK = 128
ROWS_PER_W = B // NUM_WORKERS
N_CHUNKS = ROWS_PER_W // CHUNK


def _calib_body(logits_hbm, topics_hbm, loga_hbm, b_hbm, out_hbm,
                idx_v, scale_v,
                rows0, rows1, rows2, rows3, logits0, logits1,
                sem_r0, sem_r1, sem_r2, sem_r3,
                sem_s0, sem_s1, sem_s2, sem_s3,
                sem_l0, sem_l1, sem_o0, sem_o1, sem_o2, sem_o3):
    wid = lax.axis_index("s") * NUM_CORES + lax.axis_index("c")
    base = wid * ROWS_PER_W

    rows = (rows0, rows1, rows2, rows3)
    logits_b = (logits0, logits1)
    sem_r = (sem_r0, sem_r1, sem_r2, sem_r3)
    sem_s = (sem_s0, sem_s1, sem_s2, sem_s3)
    sem_l = (sem_l0, sem_l1)
    sem_o = (sem_o0, sem_o1, sem_o2, sem_o3)

    def off(c):
        return pl.ds(pl.multiple_of(base + c * CHUNK, CHUNK), CHUNK)

    def issue_rows(c):
        return pltpu.async_copy(b_hbm.at[idx_v.at[pl.ds(c * CHUNK, CHUNK)]],
                                rows[c], sem_r[c])

    def issue_scale(c):
        return pltpu.async_copy(loga_hbm.at[idx_v.at[pl.ds(c * CHUNK, CHUNK)]],
                                scale_v.at[pl.ds(c * CHUNK, CHUNK)], sem_s[c])

    def issue_logits(c, p):
        return pltpu.async_copy(logits_hbm.at[off(c)], logits_b[p], sem_l[p])

    pltpu.sync_copy(topics_hbm.at[pl.ds(pl.multiple_of(base, CHUNK), CHUNK)],
                    idx_v.at[pl.ds(0, CHUNK)])
    rows_cp = [None] * N_CHUNKS
    scale_cp = [None] * N_CHUNKS
    logits_cp = [None] * N_CHUNKS
    rows_cp[0] = issue_rows(0)
    scale_cp[0] = issue_scale(0)
    logits_cp[0] = issue_logits(0, 0)
    pltpu.sync_copy(topics_hbm.at[pl.ds(pl.multiple_of(base + CHUNK, CHUNK),
                                        ROWS_PER_W - CHUNK)],
                    idx_v.at[pl.ds(CHUNK, ROWS_PER_W - CHUNK)])
    for c in range(1, N_CHUNKS):
        rows_cp[c] = issue_rows(c)
        scale_cp[c] = issue_scale(c)
    logits_cp[1] = issue_logits(1, 1)

    out_cp = [None] * N_CHUNKS
    for c in range(N_CHUNKS):
        p = c % 2
        scale_cp[c].wait()
        for j in range(CHUNK // LANES):
            sl = pl.ds(c * CHUNK + j * LANES, LANES)
            scale_v[sl] = jnp.exp(scale_v[sl])
        rows_cp[c].wait()
        logits_cp[c].wait()

        def group_body(g, _, c=c, p=p):
            sv = scale_v[pl.ds(c * CHUNK + g * LANES, LANES)]
            for rr in range(LANES):
                i = g * LANES + rr
                sc = sv[rr]
                for k in range(N_CLS // LANES):
                    sl = pl.ds(k * LANES, LANES)
                    plsc.addupdate(rows[c].at[i, sl], logits_b[p][i, sl] * sc)
            return 0

        lax.fori_loop(0, CHUNK // LANES, group_body, 0)
        out_cp[c] = pltpu.async_copy(rows[c], out_hbm.at[off(c)], sem_o[c])
        if c + 2 < N_CHUNKS:
            logits_cp[c + 2] = issue_logits(c + 2, p)

    for c in range(N_CHUNKS):
        out_cp[c].wait()


@jax.jit
def kernel(logits, topics, loga, b):
    topics = topics.astype(jnp.int32)
    run = pl.kernel(
        _calib_body,
        out_type=jax.ShapeDtypeStruct((B, N_CLS), jnp.float32),
        mesh=plsc.VectorSubcoreMesh(core_axis_name="c", subcore_axis_name="s"),
        scratch_types=[
            pltpu.VMEM((ROWS_PER_W,), jnp.int32),
            pltpu.VMEM((ROWS_PER_W,), jnp.float32),
        ] + [pltpu.VMEM((CHUNK, N_CLS), jnp.float32)] * 6
          + [pltpu.SemaphoreType.DMA] * 14,
    )
    return run(logits, topics, loga, b)


# speedup vs baseline: 1.3820x; 1.1461x over previous
import jax
import jax.numpy as jnp
from jax import lax
from jax.experimental import pallas as pl
from jax.experimental.pallas import tpu as pltpu
from jax.experimental.pallas import tpu_sc as plsc

N_TOP = 100000
N_CLS = 128
B = 16384

NUM_CORES = 2
NUM_SUBCORES = 16
NUM_WORKERS = NUM_CORES * NUM_SUBCORES
LANES = 16
CHUNK = 128
ROWS_PER_W = B // NUM_WORKERS
N_CHUNKS = ROWS_PER_W // CHUNK


def _calib_body(logits_hbm, topics_hbm, loga_hbm, b_hbm, out_hbm,
                idx_v, scale_v, bscale_v,
                rows0, rows1, rows2, rows3, logits0, logits1,
                sem_r0, sem_r1, sem_r2, sem_r3,
                sem_s0, sem_s1, sem_s2, sem_s3,
                sem_l0, sem_l1, sem_o0, sem_o1, sem_o2, sem_o3):
    wid = lax.axis_index("s") * NUM_CORES + lax.axis_index("c")
    base = wid * ROWS_PER_W

    rows = (rows0, rows1, rows2, rows3)
    logits_b = (logits0, logits1)
    sem_r = (sem_r0, sem_r1, sem_r2, sem_r3)
    sem_s = (sem_s0, sem_s1, sem_s2, sem_s3)
    sem_l = (sem_l0, sem_l1)
    sem_o = (sem_o0, sem_o1, sem_o2, sem_o3)

    def off(c):
        return pl.ds(pl.multiple_of(base + c * CHUNK, CHUNK), CHUNK)

    def issue_rows(c):
        return pltpu.async_copy(b_hbm.at[idx_v.at[pl.ds(c * CHUNK, CHUNK)]],
                                rows[c], sem_r[c])

    def issue_scale(c):
        return pltpu.async_copy(loga_hbm.at[idx_v.at[pl.ds(c * CHUNK, CHUNK)]],
                                scale_v.at[pl.ds(c * CHUNK, CHUNK)], sem_s[c])

    def issue_logits(c, p):
        return pltpu.async_copy(logits_hbm.at[off(c)], logits_b[p], sem_l[p])

    pltpu.sync_copy(topics_hbm.at[pl.ds(pl.multiple_of(base, CHUNK), CHUNK)],
                    idx_v.at[pl.ds(0, CHUNK)])
    rows_cp = [None] * N_CHUNKS
    scale_cp = [None] * N_CHUNKS
    logits_cp = [None] * N_CHUNKS
    rows_cp[0] = issue_rows(0)
    scale_cp[0] = issue_scale(0)
    logits_cp[0] = issue_logits(0, 0)
    pltpu.sync_copy(topics_hbm.at[pl.ds(pl.multiple_of(base + CHUNK, CHUNK),
                                        ROWS_PER_W - CHUNK)],
                    idx_v.at[pl.ds(CHUNK, ROWS_PER_W - CHUNK)])
    for c in range(1, N_CHUNKS):
        rows_cp[c] = issue_rows(c)
        scale_cp[c] = issue_scale(c)
    logits_cp[1] = issue_logits(1, 1)

    out_cp = [None] * N_CHUNKS
    for c in range(N_CHUNKS):
        p = c % 2
        scale_cp[c].wait()

        def bcast_body(g, _, c=c):
            sv = jnp.exp(scale_v[pl.ds(c * CHUNK + g * LANES, LANES)])
            for rr in range(LANES):
                bscale_v[g * LANES + rr, :] = jnp.broadcast_to(sv[rr], (LANES,))
            return 0

        lax.fori_loop(0, CHUNK // LANES, bcast_body, 0)
        rows_cp[c].wait()
        logits_cp[c].wait()

        def row_body(i, _, c=c, p=p):
            bs = bscale_v[i, :]
            for k in range(N_CLS // LANES):
                sl = pl.ds(k * LANES, LANES)
                plsc.addupdate(rows[c].at[i, sl], logits_b[p][i, sl] * bs)
            return 0

        lax.fori_loop(0, CHUNK, row_body, 0, unroll=2)
        out_cp[c] = pltpu.async_copy(rows[c], out_hbm.at[off(c)], sem_o[c])
        if c + 2 < N_CHUNKS:
            logits_cp[c + 2] = issue_logits(c + 2, p)

    for c in range(N_CHUNKS):
        out_cp[c].wait()


@jax.jit
def kernel(logits, topics, loga, b):
    topics = topics.astype(jnp.int32)
    run = pl.kernel(
        _calib_body,
        out_type=jax.ShapeDtypeStruct((B, N_CLS), jnp.float32),
        mesh=plsc.VectorSubcoreMesh(core_axis_name="c", subcore_axis_name="s"),
        scratch_types=[
            pltpu.VMEM((ROWS_PER_W,), jnp.int32),
            pltpu.VMEM((ROWS_PER_W,), jnp.float32),
            pltpu.VMEM((CHUNK, LANES), jnp.float32),
        ] + [pltpu.VMEM((CHUNK, N_CLS), jnp.float32)] * 6
          + [pltpu.SemaphoreType.DMA] * 14,
    )
    return run(logits, topics, loga, b)
